# Initial kernel scaffold; baseline (speedup 1.0000x reference)
import jax
import jax.numpy as jnp
from jax.experimental import pallas as pl


def kernel(x, msg_pass_edge_index, W_l, b_l, W_r):
    raise NotImplementedError("write your pallas kernel here")


# speedup vs baseline: 3.6707x; 3.6707x over previous
import functools

import jax
import jax.numpy as jnp
from jax import lax
from jax.experimental import pallas as pl
from jax.experimental.pallas import tpu as pltpu
from jax.experimental.pallas import tpu_sc as plsc

N_NODES = 10000
N_EDGES = 320000
D = 128
CW = 16
DA = D + CW

NC = 2
NS = 16
NW = NC * NS
K = 128
CH = 80
IB = 8
E_PAD = NW * CH * K
N_PAD = 10112
ROWS_PER_TILE = N_PAD // NS

_mesh = plsc.VectorSubcoreMesh(core_axis_name="c", subcore_axis_name="s")


@functools.partial(
    pl.kernel,
    out_type=jax.ShapeDtypeStruct((NC, N_PAD, DA), jnp.float32),
    mesh=_mesh,
    compiler_params=pltpu.CompilerParams(use_tc_tiling_on_sc=False),
    scratch_types=(
        pltpu.VMEM((IB, K), jnp.int32),
        pltpu.VMEM((IB, K), jnp.int32),
        pltpu.VMEM((K, DA), jnp.float32),
        pltpu.VMEM_SHARED((N_PAD, DA), jnp.float32),
        pltpu.SemaphoreType.DMA,
    ),
)
def _sc_aggregate(xa_hbm, src_hbm, dst_hbm, acc_out,
                  src_v, dst_v, rows_v, acc_sh, sem):
    cid = lax.axis_index("c")
    sid = lax.axis_index("s")
    wid = sid * NC + cid

    zeros16 = jnp.zeros((16,), jnp.float32)

    @pl.loop(0, K)
    def _fill(r):
        for kk in range(DA // 16):
            rows_v[r, pl.ds(kk * 16, 16)] = zeros16

    base = sid * ROWS_PER_TILE
    for t in range(ROWS_PER_TILE // K):
        pltpu.sync_copy(rows_v, acc_sh.at[pl.ds(base + t * K, K)])
    rem = ROWS_PER_TILE % K
    if rem:
        pltpu.sync_copy(rows_v.at[pl.ds(0, rem)],
                        acc_sh.at[pl.ds(base + (ROWS_PER_TILE // K) * K, rem)])
    plsc.subcore_barrier()

    @pl.loop(0, CH // IB)
    def _slab(s):
        pltpu.sync_copy(src_hbm.at[wid, pl.ds(s * IB, IB)], src_v)
        pltpu.sync_copy(dst_hbm.at[wid, pl.ds(s * IB, IB)], dst_v)

        @pl.loop(0, IB)
        def _chunk(j):
            pltpu.async_copy(xa_hbm.at[src_v.at[j]], rows_v, sem).wait()
            pltpu.sync_copy(rows_v, acc_sh.at[dst_v.at[j]], add=True)

    plsc.subcore_barrier()

    pltpu.sync_copy(acc_sh.at[pl.ds(base, ROWS_PER_TILE)],
                    acc_out.at[cid, pl.ds(base, ROWS_PER_TILE)])


_BM = 400


def _combine_body(p_ref, x_ref, wl_ref, wr_ref, b_ref, o_ref):
    p = p_ref[0] + p_ref[1]
    agg = p[:, :D]
    cnt = jnp.sum(p[:, D:], axis=1, keepdims=True) * (1.0 / CW)
    mean = agg / jnp.maximum(cnt, 1.0)
    dn = (((1,), (1,)), ((), ()))
    o_ref[...] = (
        lax.dot_general(mean, wl_ref[...], dn, preferred_element_type=jnp.float32)
        + lax.dot_general(x_ref[...], wr_ref[...], dn, preferred_element_type=jnp.float32)
        + b_ref[...]
    )


_combine = pl.pallas_call(
    _combine_body,
    grid=(N_NODES // _BM,),
    in_specs=[
        pl.BlockSpec((NC, _BM, DA), lambda j: (0, j, 0)),
        pl.BlockSpec((_BM, D), lambda j: (j, 0)),
        pl.BlockSpec((D, D), lambda j: (0, 0)),
        pl.BlockSpec((D, D), lambda j: (0, 0)),
        pl.BlockSpec((1, D), lambda j: (0, 0)),
    ],
    out_specs=pl.BlockSpec((_BM, D), lambda j: (j, 0)),
    out_shape=jax.ShapeDtypeStruct((N_NODES, D), jnp.float32),
)


def kernel(x, msg_pass_edge_index, W_l, b_l, W_r):
    src = msg_pass_edge_index[0].astype(jnp.int32)
    dst = msg_pass_edge_index[1].astype(jnp.int32)
    pad = E_PAD - N_EDGES
    src_p = jnp.concatenate([src, jnp.zeros((pad,), jnp.int32)]).reshape(NW, CH, K)
    dst_p = jnp.concatenate([dst, jnp.full((pad,), N_NODES, jnp.int32)]).reshape(NW, CH, K)
    xa = jnp.concatenate([x, jnp.ones((N_NODES, CW), jnp.float32)], axis=1)
    acc_p = _sc_aggregate(xa, src_p, dst_p)
    return _combine(acc_p, x, W_l, W_r, b_l.reshape(1, D))


# speedup vs baseline: 3.6747x; 1.0011x over previous
import functools

import jax
import jax.numpy as jnp
from jax import lax
from jax.experimental import pallas as pl
from jax.experimental.pallas import tpu as pltpu
from jax.experimental.pallas import tpu_sc as plsc

N_NODES = 10000
N_EDGES = 320000
D = 128
CW = 16
DA = D + CW

NC = 2
NS = 16
NW = NC * NS
K = 128
CH = 80
IB = 8
E_PAD = NW * CH * K
N_PAD = 10112
ROWS_PER_TILE = N_PAD // NS

_mesh = plsc.VectorSubcoreMesh(core_axis_name="c", subcore_axis_name="s")


@functools.partial(
    pl.kernel,
    out_type=jax.ShapeDtypeStruct((NC, N_PAD, DA), jnp.float32),
    mesh=_mesh,
    compiler_params=pltpu.CompilerParams(use_tc_tiling_on_sc=False),
    scratch_types=(
        pltpu.VMEM((IB, K), jnp.int32),
        pltpu.VMEM((IB, K), jnp.int32),
        pltpu.VMEM((K, DA), jnp.float32),
        pltpu.VMEM_SHARED((N_PAD, DA), jnp.float32),
        pltpu.SemaphoreType.DMA,
    ),
)
def _sc_aggregate(xa_hbm, src_hbm, dst_hbm, acc_out,
                  src_v, dst_v, rows_v, acc_sh, sem):
    cid = lax.axis_index("c")
    sid = lax.axis_index("s")
    wid = sid * NC + cid

    zeros16 = jnp.zeros((16,), jnp.float32)

    @pl.loop(0, K)
    def _fill(r):
        for kk in range(DA // 16):
            rows_v[r, pl.ds(kk * 16, 16)] = zeros16

    base = sid * ROWS_PER_TILE
    for t in range(ROWS_PER_TILE // K):
        pltpu.sync_copy(rows_v, acc_sh.at[pl.ds(base + t * K, K)])
    rem = ROWS_PER_TILE % K
    if rem:
        pltpu.sync_copy(rows_v.at[pl.ds(0, rem)],
                        acc_sh.at[pl.ds(base + (ROWS_PER_TILE // K) * K, rem)])
    plsc.subcore_barrier()

    @pl.loop(0, CH // IB)
    def _slab(s):
        pltpu.sync_copy(src_hbm.at[wid, pl.ds(s * IB, IB)], src_v)
        pltpu.sync_copy(dst_hbm.at[wid, pl.ds(s * IB, IB)], dst_v)

        @pl.loop(0, IB)
        def _chunk(j):
            pltpu.async_copy(xa_hbm.at[src_v.at[j]], rows_v, sem).wait()
            pltpu.sync_copy(rows_v, acc_sh.at[dst_v.at[j]], add=True)

    plsc.subcore_barrier()

    pltpu.sync_copy(acc_sh.at[pl.ds(base, ROWS_PER_TILE)],
                    acc_out.at[cid, pl.ds(base, ROWS_PER_TILE)])


_BM = 400


def _combine_body(p_ref, x_ref, wl_ref, wr_ref, b_ref, o_ref):
    p = p_ref[0] + p_ref[1]
    agg = p[:, :D]
    cnt = jnp.sum(p[:, D:], axis=1, keepdims=True) * (1.0 / CW)
    mean = agg / jnp.maximum(cnt, 1.0)
    dn = (((1,), (1,)), ((), ()))
    o_ref[...] = (
        lax.dot_general(mean, wl_ref[...], dn, preferred_element_type=jnp.float32)
        + lax.dot_general(x_ref[...], wr_ref[...], dn, preferred_element_type=jnp.float32)
        + b_ref[...]
    )


_combine = pl.pallas_call(
    _combine_body,
    grid=(N_NODES // _BM,),
    in_specs=[
        pl.BlockSpec((NC, _BM, DA), lambda j: (0, j, 0)),
        pl.BlockSpec((_BM, D), lambda j: (j, 0)),
        pl.BlockSpec((D, D), lambda j: (0, 0)),
        pl.BlockSpec((D, D), lambda j: (0, 0)),
        pl.BlockSpec((1, D), lambda j: (0, 0)),
    ],
    out_specs=pl.BlockSpec((_BM, D), lambda j: (j, 0)),
    out_shape=jax.ShapeDtypeStruct((N_NODES, D), jnp.float32),
)


def kernel(x, msg_pass_edge_index, W_l, b_l, W_r):
    src = msg_pass_edge_index[0].astype(jnp.int32)
    dst = msg_pass_edge_index[1].astype(jnp.int32)
    pad = E_PAD - N_EDGES
    trash = N_NODES + jnp.arange(pad, dtype=jnp.int32) % (N_PAD - N_NODES)
    src_p = jnp.concatenate([src, jnp.zeros((pad,), jnp.int32)]).reshape(NW, CH, K)
    dst_p = jnp.concatenate([dst, trash]).reshape(NW, CH, K)
    xa = jnp.concatenate([x, jnp.ones((N_NODES, CW), jnp.float32)], axis=1)
    acc_p = _sc_aggregate(xa, src_p, dst_p)
    return _combine(acc_p, x, W_l, W_r, b_l.reshape(1, D))
